# baseline (device time: 389023 ns/iter reference)
import jax
import jax.numpy as jnp
from jax import lax
from jax.experimental import pallas as pl
from jax.experimental.pallas import tpu as pltpu

N_Z = 4
SCALE = 64 ** -0.5


def _partials_body(q_ref, k_ref, v_ref, m_ref, l_ref, o_ref, *, h, d):
    q = q_ref[0, 0]
    k4 = k_ref[0]
    v4 = v_ref[0]

    s_cols = []
    for j in range(h):
        qj = (q[j, :] * SCALE).reshape(d, 1)
        s_cols.append(jnp.dot(k4[:, j, :], qj,
                              preferred_element_type=jnp.float32))
    s = jnp.concatenate(s_cols, axis=1)

    m = jnp.max(s, axis=0)
    p = jnp.exp(s - m[None, :])
    l = jnp.sum(p, axis=0)

    pt = p.T
    o_rows = []
    for j in range(h):
        o_rows.append(jnp.dot(pt[j:j + 1, :], v4[:, j, :],
                              preferred_element_type=jnp.float32))
    o = jnp.concatenate(o_rows, axis=0)

    m_ref[0, 0, :] = m
    l_ref[0, 0, :] = l
    o_ref[0] = o


def _local_partials(Q, K, V):
    import functools
    b, kk, h, d = K.shape
    return pl.pallas_call(
        functools.partial(_partials_body, h=h, d=d),
        grid=(b,),
        in_specs=[
            pl.BlockSpec((1, 1, h, d), lambda i: (i, 0, 0, 0)),
            pl.BlockSpec((1, kk, h, d), lambda i: (i, 0, 0, 0)),
            pl.BlockSpec((1, kk, h, d), lambda i: (i, 0, 0, 0)),
        ],
        out_specs=[
            pl.BlockSpec((1, 1, h), lambda i: (i, 0, 0)),
            pl.BlockSpec((1, 1, h), lambda i: (i, 0, 0)),
            pl.BlockSpec((1, h, d), lambda i: (i, 0, 0)),
        ],
        out_shape=[
            jax.ShapeDtypeStruct((b, 1, h), jnp.float32),
            jax.ShapeDtypeStruct((b, 1, h), jnp.float32),
            jax.ShapeDtypeStruct((b, h, d), jnp.float32),
        ],
        compiler_params=pltpu.CompilerParams(
            vmem_limit_bytes=100 * 1024 * 1024,
        ),
    )(Q, K, V)


def _combine_body(m_ref, l_ref, o_ref, out_ref,
                  cm_ref, cl_ref, co_ref, send_sems, recv_sems):
    my_x = lax.axis_index("x")
    my_y = lax.axis_index("y")
    my_z = lax.axis_index("z")

    tensors = ((m_ref, cm_ref), (l_ref, cl_ref), (o_ref, co_ref))

    cm_ref[pl.ds(my_z, 1)] = m_ref[...][None]
    cl_ref[pl.ds(my_z, 1)] = l_ref[...][None]
    co_ref[pl.ds(my_z, 1)] = o_ref[...][None]

    sends = []
    for dz in range(1, N_Z):
        zz = lax.rem(my_z + dz, N_Z)
        for ti, (src, dst) in enumerate(tensors):
            rdma = pltpu.make_async_remote_copy(
                src_ref=src,
                dst_ref=dst.at[my_z],
                send_sem=send_sems.at[dz - 1, ti],
                recv_sem=recv_sems.at[my_z, ti],
                device_id=(my_x, my_y, zz),
                device_id_type=pl.DeviceIdType.MESH,
            )
            rdma.start()
            sends.append(rdma)

    for dz in range(1, N_Z):
        src_z = lax.rem(my_z + dz, N_Z)
        for ti, (src, dst) in enumerate(tensors):
            rdma = pltpu.make_async_remote_copy(
                src_ref=src,
                dst_ref=dst.at[src_z],
                send_sem=send_sems.at[dz - 1, ti],
                recv_sem=recv_sems.at[src_z, ti],
                device_id=(my_x, my_y, src_z),
                device_id_type=pl.DeviceIdType.MESH,
            )
            rdma.wait_recv()

    cm = cm_ref[...][:, :, 0, :]
    cl = cl_ref[...][:, :, 0, :]
    co = co_ref[...]
    g_m = jnp.max(cm, axis=0)
    alpha = jnp.exp(cm - g_m[None])
    g_l = jnp.sum(cl * alpha, axis=0)
    o = jnp.sum(co * alpha[..., None], axis=0) / g_l[..., None]
    out_ref[...] = o[:, None]

    for rdma in sends:
        rdma.wait_send()


def _combine(m, l, o):
    b, h = m.shape[0], m.shape[-1]
    d = o.shape[-1]
    return pl.pallas_call(
        _combine_body,
        in_specs=[
            pl.BlockSpec(memory_space=pltpu.VMEM),
            pl.BlockSpec(memory_space=pltpu.VMEM),
            pl.BlockSpec(memory_space=pltpu.VMEM),
        ],
        out_specs=pl.BlockSpec(memory_space=pltpu.VMEM),
        out_shape=jax.ShapeDtypeStruct((b, 1, h, d), jnp.float32),
        scratch_shapes=[
            pltpu.VMEM((N_Z, b, 1, h), jnp.float32),
            pltpu.VMEM((N_Z, b, 1, h), jnp.float32),
            pltpu.VMEM((N_Z, b, h, d), jnp.float32),
            pltpu.SemaphoreType.DMA((N_Z - 1, 3)),
            pltpu.SemaphoreType.DMA((N_Z, 3)),
        ],
        compiler_params=pltpu.CompilerParams(has_side_effects=True),
    )(m, l, o)


def kernel(Q, K, V):
    m, l, o = _local_partials(Q, K, V)
    return _combine(m, l, o)


# device time: 342035 ns/iter; 1.1374x vs baseline; 1.1374x over previous
import jax
import jax.numpy as jnp
from jax import lax
from jax.experimental import pallas as pl
from jax.experimental.pallas import tpu as pltpu

N_Z = 4
SCALE = 64 ** -0.5


def _partials_body(q_ref, k_ref, v_ref, m_ref, l_ref, o_ref, *, h, d):
    kk = k_ref.shape[1]
    q = q_ref[0, 0]
    k2 = k_ref[0].reshape(kk * h, d)
    v2 = v_ref[0].reshape(kk * h, d)

    qt = (q * SCALE).T
    g = jnp.dot(k2, qt, preferred_element_type=jnp.float32)
    g3 = g.reshape(kk, h, h)
    sel3 = (lax.broadcasted_iota(jnp.int32, (kk, h, h), 1)
            == lax.broadcasted_iota(jnp.int32, (kk, h, h), 2))
    s = jnp.sum(jnp.where(sel3, g3, 0.0), axis=2)

    m = jnp.max(s, axis=0)
    p = jnp.exp(s - m[None, :])
    l = jnp.sum(p, axis=0)

    pflat = p.reshape(1, kk * h)
    hsel = (lax.broadcasted_iota(jnp.int32, (h, kk * h), 0)
            == lax.broadcasted_iota(jnp.int32, (h, kk * h), 1) % h)
    a = jnp.where(hsel, pflat, 0.0)
    o = jnp.dot(a, v2, preferred_element_type=jnp.float32)

    m_ref[0, 0, :] = m
    l_ref[0, 0, :] = l
    o_ref[0] = o


def _local_partials(Q, K, V):
    import functools
    b, kk, h, d = K.shape
    return pl.pallas_call(
        functools.partial(_partials_body, h=h, d=d),
        grid=(b,),
        in_specs=[
            pl.BlockSpec((1, 1, h, d), lambda i: (i, 0, 0, 0)),
            pl.BlockSpec((1, kk, h, d), lambda i: (i, 0, 0, 0)),
            pl.BlockSpec((1, kk, h, d), lambda i: (i, 0, 0, 0)),
        ],
        out_specs=[
            pl.BlockSpec((1, 1, h), lambda i: (i, 0, 0)),
            pl.BlockSpec((1, 1, h), lambda i: (i, 0, 0)),
            pl.BlockSpec((1, h, d), lambda i: (i, 0, 0)),
        ],
        out_shape=[
            jax.ShapeDtypeStruct((b, 1, h), jnp.float32),
            jax.ShapeDtypeStruct((b, 1, h), jnp.float32),
            jax.ShapeDtypeStruct((b, h, d), jnp.float32),
        ],
        compiler_params=pltpu.CompilerParams(
            vmem_limit_bytes=100 * 1024 * 1024,
        ),
    )(Q, K, V)


def _combine_body(m_ref, l_ref, o_ref, out_ref,
                  cm_ref, cl_ref, co_ref, send_sems, recv_sems):
    my_x = lax.axis_index("x")
    my_y = lax.axis_index("y")
    my_z = lax.axis_index("z")

    tensors = ((m_ref, cm_ref), (l_ref, cl_ref), (o_ref, co_ref))

    cm_ref[pl.ds(my_z, 1)] = m_ref[...][None]
    cl_ref[pl.ds(my_z, 1)] = l_ref[...][None]
    co_ref[pl.ds(my_z, 1)] = o_ref[...][None]

    sends = []
    for dz in range(1, N_Z):
        zz = lax.rem(my_z + dz, N_Z)
        for ti, (src, dst) in enumerate(tensors):
            rdma = pltpu.make_async_remote_copy(
                src_ref=src,
                dst_ref=dst.at[my_z],
                send_sem=send_sems.at[dz - 1, ti],
                recv_sem=recv_sems.at[my_z, ti],
                device_id=(my_x, my_y, zz),
                device_id_type=pl.DeviceIdType.MESH,
            )
            rdma.start()
            sends.append(rdma)

    for dz in range(1, N_Z):
        src_z = lax.rem(my_z + dz, N_Z)
        for ti, (src, dst) in enumerate(tensors):
            rdma = pltpu.make_async_remote_copy(
                src_ref=src,
                dst_ref=dst.at[src_z],
                send_sem=send_sems.at[dz - 1, ti],
                recv_sem=recv_sems.at[src_z, ti],
                device_id=(my_x, my_y, src_z),
                device_id_type=pl.DeviceIdType.MESH,
            )
            rdma.wait_recv()

    cm = cm_ref[...][:, :, 0, :]
    cl = cl_ref[...][:, :, 0, :]
    co = co_ref[...]
    g_m = jnp.max(cm, axis=0)
    alpha = jnp.exp(cm - g_m[None])
    g_l = jnp.sum(cl * alpha, axis=0)
    o = jnp.sum(co * alpha[..., None], axis=0) / g_l[..., None]
    out_ref[...] = o[:, None]

    for rdma in sends:
        rdma.wait_send()


def _combine(m, l, o):
    b, h = m.shape[0], m.shape[-1]
    d = o.shape[-1]
    return pl.pallas_call(
        _combine_body,
        in_specs=[
            pl.BlockSpec(memory_space=pltpu.VMEM),
            pl.BlockSpec(memory_space=pltpu.VMEM),
            pl.BlockSpec(memory_space=pltpu.VMEM),
        ],
        out_specs=pl.BlockSpec(memory_space=pltpu.VMEM),
        out_shape=jax.ShapeDtypeStruct((b, 1, h, d), jnp.float32),
        scratch_shapes=[
            pltpu.VMEM((N_Z, b, 1, h), jnp.float32),
            pltpu.VMEM((N_Z, b, 1, h), jnp.float32),
            pltpu.VMEM((N_Z, b, h, d), jnp.float32),
            pltpu.SemaphoreType.DMA((N_Z - 1, 3)),
            pltpu.SemaphoreType.DMA((N_Z, 3)),
        ],
        compiler_params=pltpu.CompilerParams(has_side_effects=True),
    )(m, l, o)


def kernel(Q, K, V):
    m, l, o = _local_partials(Q, K, V)
    return _combine(m, l, o)


# device time: 70178 ns/iter; 5.5434x vs baseline; 4.8738x over previous
import functools

import jax
import jax.numpy as jnp
from jax import lax
from jax.experimental import pallas as pl
from jax.experimental.pallas import tpu as pltpu

N_Z = 4
SCALE = 64 ** -0.5


def _partials_body(q_ref, k_ref, v_ref, m_ref, l_ref, o_ref, *, h, d):
    kk = k_ref.shape[-1]
    hd = h * d
    q = q_ref[0, 0]
    k2 = k_ref[0].reshape(hd, kk)
    v2 = v_ref[0].reshape(hd, kk)

    qflat = (q * SCALE).reshape(1, hd)
    qsel = (lax.broadcasted_iota(jnp.int32, (h, hd), 0)
            == lax.broadcasted_iota(jnp.int32, (h, hd), 1) // d)
    q2t = jnp.where(qsel, qflat, 0.0)

    st = jnp.dot(q2t, k2, preferred_element_type=jnp.float32)
    m = jnp.max(st, axis=1)
    p = jnp.exp(st - m[:, None])
    l = jnp.sum(p, axis=1)

    pt = p.T
    o_cross = jnp.dot(v2, pt, preferred_element_type=jnp.float32)
    osel = (lax.broadcasted_iota(jnp.int32, (hd, h), 0) // d
            == lax.broadcasted_iota(jnp.int32, (hd, h), 1))
    o_flat = jnp.sum(jnp.where(osel, o_cross, 0.0), axis=1)

    m_ref[0, 0, :] = m
    l_ref[0, 0, :] = l
    o_ref[0, 0, :] = o_flat


def _local_partials(Q, K, V):
    b, kk, h, d = K.shape
    KT = jnp.transpose(K, (0, 2, 3, 1))
    VT = jnp.transpose(V, (0, 2, 3, 1))
    Q3 = Q.reshape(b, 1, h * d)
    return pl.pallas_call(
        functools.partial(_partials_body, h=h, d=d),
        grid=(b,),
        in_specs=[
            pl.BlockSpec((1, 1, h * d), lambda i: (i, 0, 0)),
            pl.BlockSpec((1, h, d, kk), lambda i: (i, 0, 0, 0)),
            pl.BlockSpec((1, h, d, kk), lambda i: (i, 0, 0, 0)),
        ],
        out_specs=[
            pl.BlockSpec((1, 1, h), lambda i: (i, 0, 0)),
            pl.BlockSpec((1, 1, h), lambda i: (i, 0, 0)),
            pl.BlockSpec((1, 1, h * d), lambda i: (i, 0, 0)),
        ],
        out_shape=[
            jax.ShapeDtypeStruct((b, 1, h), jnp.float32),
            jax.ShapeDtypeStruct((b, 1, h), jnp.float32),
            jax.ShapeDtypeStruct((b, 1, h * d), jnp.float32),
        ],
        compiler_params=pltpu.CompilerParams(
            vmem_limit_bytes=100 * 1024 * 1024,
        ),
    )(Q3, KT, VT)


def _combine_body(m_ref, l_ref, o_ref, out_ref,
                  cm_ref, cl_ref, co_ref, send_sems, recv_sems, *, h, d):
    my_x = lax.axis_index("x")
    my_y = lax.axis_index("y")
    my_z = lax.axis_index("z")

    tensors = ((m_ref, cm_ref), (l_ref, cl_ref), (o_ref, co_ref))

    cm_ref[pl.ds(my_z, 1)] = m_ref[...][None]
    cl_ref[pl.ds(my_z, 1)] = l_ref[...][None]
    co_ref[pl.ds(my_z, 1)] = o_ref[...][None]

    sends = []
    for dz in range(1, N_Z):
        zz = lax.rem(my_z + dz, N_Z)
        for ti, (src, dst) in enumerate(tensors):
            rdma = pltpu.make_async_remote_copy(
                src_ref=src,
                dst_ref=dst.at[my_z],
                send_sem=send_sems.at[dz - 1, ti],
                recv_sem=recv_sems.at[my_z, ti],
                device_id=(my_x, my_y, zz),
                device_id_type=pl.DeviceIdType.MESH,
            )
            rdma.start()
            sends.append(rdma)

    for dz in range(1, N_Z):
        src_z = lax.rem(my_z + dz, N_Z)
        for ti, (src, dst) in enumerate(tensors):
            rdma = pltpu.make_async_remote_copy(
                src_ref=src,
                dst_ref=dst.at[src_z],
                send_sem=send_sems.at[dz - 1, ti],
                recv_sem=recv_sems.at[src_z, ti],
                device_id=(my_x, my_y, src_z),
                device_id_type=pl.DeviceIdType.MESH,
            )
            rdma.wait_recv()

    b = m_ref.shape[0]
    cm = cm_ref[...][:, :, 0, :]
    cl = cl_ref[...][:, :, 0, :]
    co = co_ref[...]
    g_m = jnp.max(cm, axis=0)
    alpha = jnp.exp(cm - g_m[None])
    g_l = jnp.sum(cl * alpha, axis=0)
    alpha_f = jnp.broadcast_to(
        alpha[..., None], (N_Z, b, h, d)).reshape(N_Z, b, 1, h * d)
    g_l_f = jnp.broadcast_to(
        g_l[..., None], (b, h, d)).reshape(b, 1, h * d)
    o = jnp.sum(co * alpha_f, axis=0) / g_l_f
    out_ref[...] = o.reshape(b, 1, h, d)

    for rdma in sends:
        rdma.wait_send()


def _combine(m, l, o, h, d):
    b = m.shape[0]
    return pl.pallas_call(
        functools.partial(_combine_body, h=h, d=d),
        in_specs=[
            pl.BlockSpec(memory_space=pltpu.VMEM),
            pl.BlockSpec(memory_space=pltpu.VMEM),
            pl.BlockSpec(memory_space=pltpu.VMEM),
        ],
        out_specs=pl.BlockSpec(memory_space=pltpu.VMEM),
        out_shape=jax.ShapeDtypeStruct((b, 1, h, d), jnp.float32),
        scratch_shapes=[
            pltpu.VMEM((N_Z, b, 1, h), jnp.float32),
            pltpu.VMEM((N_Z, b, 1, h), jnp.float32),
            pltpu.VMEM((N_Z, b, 1, h * d), jnp.float32),
            pltpu.SemaphoreType.DMA((N_Z - 1, 3)),
            pltpu.SemaphoreType.DMA((N_Z, 3)),
        ],
        compiler_params=pltpu.CompilerParams(has_side_effects=True),
    )(m, l, o)


def kernel(Q, K, V):
    b, kk, h, d = K.shape
    m, l, o = _local_partials(Q, K, V)
    return _combine(m, l, o, h, d)


# device time: 63193 ns/iter; 6.1561x vs baseline; 1.1105x over previous
import functools

import jax
import jax.numpy as jnp
from jax import lax
from jax.experimental import pallas as pl
from jax.experimental.pallas import tpu as pltpu

N_Z = 4
SCALE = 64 ** -0.5


def _partials_body(q_ref, k_ref, v_ref, o_ref, *, h, d):
    kk = k_ref.shape[-1]
    hd = h * d
    q = q_ref[0, 0]
    k2 = k_ref[0].reshape(hd, kk)
    v2 = v_ref[0].reshape(hd, kk)

    qflat = (q * SCALE).reshape(1, hd)
    qsel = (lax.broadcasted_iota(jnp.int32, (h, hd), 0)
            == lax.broadcasted_iota(jnp.int32, (h, hd), 1) // d)
    q2t = jnp.where(qsel, qflat, 0.0)

    st = jnp.dot(q2t, k2, preferred_element_type=jnp.float32)
    m = jnp.max(st, axis=1)
    p = jnp.exp(st - m[:, None])
    l = jnp.sum(p, axis=1)

    pt = p.T
    o_cross = jnp.dot(v2, pt, preferred_element_type=jnp.float32)
    osel = (lax.broadcasted_iota(jnp.int32, (hd, h), 0) // d
            == lax.broadcasted_iota(jnp.int32, (hd, h), 1))
    o_flat = jnp.sum(jnp.where(osel, o_cross, 0.0), axis=1)

    o_ref[0, 0, :hd] = o_flat
    o_ref[0, 0, hd:hd + h] = m
    o_ref[0, 0, hd + h:] = l


def _local_partials(Q, K, V):
    b, kk, h, d = K.shape
    KT = jnp.transpose(K, (0, 2, 3, 1))
    VT = jnp.transpose(V, (0, 2, 3, 1))
    Q3 = Q.reshape(b, 1, h * d)
    return pl.pallas_call(
        functools.partial(_partials_body, h=h, d=d),
        grid=(b,),
        in_specs=[
            pl.BlockSpec((1, 1, h * d), lambda i: (i, 0, 0)),
            pl.BlockSpec((1, h, d, kk), lambda i: (i, 0, 0, 0)),
            pl.BlockSpec((1, h, d, kk), lambda i: (i, 0, 0, 0)),
        ],
        out_specs=pl.BlockSpec((1, 1, h * d + 2 * h), lambda i: (i, 0, 0)),
        out_shape=jax.ShapeDtypeStruct((b, 1, h * d + 2 * h), jnp.float32),
        compiler_params=pltpu.CompilerParams(
            vmem_limit_bytes=100 * 1024 * 1024,
        ),
    )(Q3, KT, VT)


def _combine_body(p_ref, out_ref, comm_ref, send_sems, recv_sems, *, h, d):
    my_x = lax.axis_index("x")
    my_y = lax.axis_index("y")
    my_z = lax.axis_index("z")
    hd = h * d

    barrier_sem = pltpu.get_barrier_semaphore()
    for dz in range(1, N_Z):
        pl.semaphore_signal(
            barrier_sem, inc=1,
            device_id=(my_x, my_y, lax.rem(my_z + dz, N_Z)),
            device_id_type=pl.DeviceIdType.MESH,
        )
    pl.semaphore_wait(barrier_sem, N_Z - 1)

    comm_ref[pl.ds(my_z, 1)] = p_ref[...][None]

    sends = []
    for dz in range(1, N_Z):
        zz = lax.rem(my_z + dz, N_Z)
        rdma = pltpu.make_async_remote_copy(
            src_ref=p_ref,
            dst_ref=comm_ref.at[my_z],
            send_sem=send_sems.at[dz - 1],
            recv_sem=recv_sems.at[my_z],
            device_id=(my_x, my_y, zz),
            device_id_type=pl.DeviceIdType.MESH,
        )
        rdma.start()
        sends.append(rdma)

    for dz in range(1, N_Z):
        src_z = lax.rem(my_z + dz, N_Z)
        rdma = pltpu.make_async_remote_copy(
            src_ref=p_ref,
            dst_ref=comm_ref.at[src_z],
            send_sem=send_sems.at[dz - 1],
            recv_sem=recv_sems.at[src_z],
            device_id=(my_x, my_y, src_z),
            device_id_type=pl.DeviceIdType.MESH,
        )
        rdma.wait_recv()

    b = p_ref.shape[0]
    c = comm_ref[...]
    cm = c[:, :, 0, hd:hd + h]
    cl = c[:, :, 0, hd + h:]
    co = c[..., :hd]
    g_m = jnp.max(cm, axis=0)
    alpha = jnp.exp(cm - g_m[None])
    g_l = jnp.sum(cl * alpha, axis=0)
    alpha_f = jnp.broadcast_to(
        alpha[..., None], (N_Z, b, h, d)).reshape(N_Z, b, 1, h * d)
    g_l_f = jnp.broadcast_to(
        g_l[..., None], (b, h, d)).reshape(b, 1, h * d)
    o = jnp.sum(co * alpha_f, axis=0) / g_l_f
    out_ref[...] = o.reshape(b, 1, h, d)

    for rdma in sends:
        rdma.wait_send()


def _combine(p, h, d):
    b = p.shape[0]
    return pl.pallas_call(
        functools.partial(_combine_body, h=h, d=d),
        in_specs=[pl.BlockSpec(memory_space=pltpu.VMEM)],
        out_specs=pl.BlockSpec(memory_space=pltpu.VMEM),
        out_shape=jax.ShapeDtypeStruct((b, 1, h, d), jnp.float32),
        scratch_shapes=[
            pltpu.VMEM((N_Z, b, 1, h * d + 2 * h), jnp.float32),
            pltpu.SemaphoreType.DMA((N_Z - 1,)),
            pltpu.SemaphoreType.DMA((N_Z,)),
        ],
        compiler_params=pltpu.CompilerParams(
            has_side_effects=True,
            collective_id=0,
        ),
    )(p)


def kernel(Q, K, V):
    b, kk, h, d = K.shape
    p = _local_partials(Q, K, V)
    return _combine(p, h, d)


# device time: 54340 ns/iter; 7.1591x vs baseline; 1.1629x over previous
import functools

import jax
import jax.numpy as jnp
from jax import lax
from jax.experimental import pallas as pl
from jax.experimental.pallas import tpu as pltpu

N_Z = 4
SCALE = 64 ** -0.5


def _body(q_ref, k_ref, v_ref, out_ref, comm_ref, send_sems, recv_sems,
          *, b, h, d):
    kk = k_ref.shape[-1]
    hd = h * d
    row_w = hd + 2 * h
    i = pl.program_id(0)

    my_x = lax.axis_index("x")
    my_y = lax.axis_index("y")
    my_z = lax.axis_index("z")

    @pl.when(i == 0)
    def _():
        barrier_sem = pltpu.get_barrier_semaphore()
        for dz in range(1, N_Z):
            pl.semaphore_signal(
                barrier_sem, inc=1,
                device_id=(my_x, my_y, lax.rem(my_z + dz, N_Z)),
                device_id_type=pl.DeviceIdType.MESH,
            )
        pl.semaphore_wait(barrier_sem, N_Z - 1)

    q = q_ref[0, 0]
    k2 = k_ref[0].reshape(hd, kk)
    v2 = v_ref[0].reshape(hd, kk)

    qflat = (q * SCALE).reshape(1, hd)
    qsel = (lax.broadcasted_iota(jnp.int32, (h, hd), 0)
            == lax.broadcasted_iota(jnp.int32, (h, hd), 1) // d)
    q2t = jnp.where(qsel, qflat, 0.0)

    st = jnp.dot(q2t, k2, preferred_element_type=jnp.float32)
    m = jnp.max(st, axis=1)
    p = jnp.exp(st - m[:, None])
    l = jnp.sum(p, axis=1)

    pt = p.T
    o_cross = jnp.dot(v2, pt, preferred_element_type=jnp.float32)
    osel = (lax.broadcasted_iota(jnp.int32, (hd, h), 0) // d
            == lax.broadcasted_iota(jnp.int32, (hd, h), 1))
    o_flat = jnp.sum(jnp.where(osel, o_cross, 0.0), axis=1)

    row = jnp.concatenate([o_flat[None, :], m[None, :], l[None, :]], axis=1)
    comm_ref[pl.ds(my_z, 1), pl.ds(i, 1)] = row.reshape(1, 1, 1, row_w)

    for dz in range(1, N_Z):
        zz = lax.rem(my_z + dz, N_Z)
        rdma = pltpu.make_async_remote_copy(
            src_ref=comm_ref.at[my_z, i],
            dst_ref=comm_ref.at[my_z, i],
            send_sem=send_sems.at[dz - 1, i],
            recv_sem=recv_sems.at[my_z, i],
            device_id=(my_x, my_y, zz),
            device_id_type=pl.DeviceIdType.MESH,
        )
        rdma.start()

    @pl.when(i == b - 1)
    def _():
        for dz in range(1, N_Z):
            src_z = lax.rem(my_z + dz, N_Z)
            for j in range(b):
                rdma = pltpu.make_async_remote_copy(
                    src_ref=comm_ref.at[src_z, j],
                    dst_ref=comm_ref.at[src_z, j],
                    send_sem=send_sems.at[dz - 1, j],
                    recv_sem=recv_sems.at[src_z, j],
                    device_id=(my_x, my_y, src_z),
                    device_id_type=pl.DeviceIdType.MESH,
                )
                rdma.wait_recv()

        c = comm_ref[...]
        cm = c[:, :, 0, hd:hd + h]
        cl = c[:, :, 0, hd + h:]
        co = c[..., :hd]
        g_m = jnp.max(cm, axis=0)
        alpha = jnp.exp(cm - g_m[None])
        g_l = jnp.sum(cl * alpha, axis=0)
        alpha_f = jnp.broadcast_to(
            alpha[..., None], (N_Z, b, h, d)).reshape(N_Z, b, 1, hd)
        g_l_f = jnp.broadcast_to(
            g_l[..., None], (b, h, d)).reshape(b, 1, hd)
        o = jnp.sum(co * alpha_f, axis=0) / g_l_f
        out_ref[...] = o.reshape(b, 1, h, d)

        for dz in range(1, N_Z):
            for j in range(b):
                rdma = pltpu.make_async_remote_copy(
                    src_ref=comm_ref.at[my_z, j],
                    dst_ref=comm_ref.at[my_z, j],
                    send_sem=send_sems.at[dz - 1, j],
                    recv_sem=recv_sems.at[my_z, j],
                    device_id=(my_x, my_y, lax.rem(my_z + dz, N_Z)),
                    device_id_type=pl.DeviceIdType.MESH,
                )
                rdma.wait_send()


def kernel(Q, K, V):
    b, kk, h, d = K.shape
    KT = jnp.transpose(K, (0, 2, 3, 1))
    VT = jnp.transpose(V, (0, 2, 3, 1))
    Q3 = Q.reshape(b, 1, h * d)
    row_w = h * d + 2 * h
    return pl.pallas_call(
        functools.partial(_body, b=b, h=h, d=d),
        grid=(b,),
        in_specs=[
            pl.BlockSpec((1, 1, h * d), lambda i: (i, 0, 0)),
            pl.BlockSpec((1, h, d, kk), lambda i: (i, 0, 0, 0)),
            pl.BlockSpec((1, h, d, kk), lambda i: (i, 0, 0, 0)),
        ],
        out_specs=pl.BlockSpec((b, 1, h, d), lambda i: (0, 0, 0, 0)),
        out_shape=jax.ShapeDtypeStruct((b, 1, h, d), jnp.float32),
        scratch_shapes=[
            pltpu.VMEM((N_Z, b, 1, row_w), jnp.float32),
            pltpu.SemaphoreType.DMA((N_Z - 1, b)),
            pltpu.SemaphoreType.DMA((N_Z, b)),
        ],
        compiler_params=pltpu.CompilerParams(
            has_side_effects=True,
            collective_id=0,
            vmem_limit_bytes=100 * 1024 * 1024,
        ),
    )(Q3, KT, VT)
